# baseline (speedup 1.0000x reference)
import functools

import jax
import jax.numpy as jnp
from jax import lax
from jax.experimental import pallas as pl
from jax.experimental.pallas import tpu as pltpu
from jax.experimental.pallas import tpu_sc as plsc

NBITS = 16
HIDDEN = 128
BATCH = 32
T = 8192

NUM_CORES = 2
NUM_SUBCORES = 16
LANES = 16

B_BLK = 8
T_BLK = 2048


def _sc_bias_body(msg_hbm, emb_hbm, bias_hbm, msg_v, idx_v, rows_v, acc_v, sem):
    wid = lax.axis_index("s") * NUM_CORES + lax.axis_index("c")
    pltpu.sync_copy(msg_hbm.at[wid], msg_v)
    idx_v[...] = 2 * lax.iota(jnp.int32, LANES) + msg_v[...]
    pltpu.async_copy(emb_hbm.at[idx_v], rows_v, sem).wait()
    for h in range(HIDDEN // LANES):
        acc = rows_v[0, pl.ds(h * LANES, LANES)]
        for i in range(1, NBITS):
            acc = acc + rows_v[i, pl.ds(h * LANES, LANES)]
        acc_v[pl.ds(h * LANES, LANES)] = acc
    pltpu.sync_copy(acc_v, bias_hbm.at[wid])


_sc_bias = functools.partial(
    pl.kernel,
    mesh=plsc.VectorSubcoreMesh(core_axis_name="c", subcore_axis_name="s"),
    out_type=jax.ShapeDtypeStruct((BATCH, HIDDEN), jnp.float32),
    scratch_types=[
        pltpu.VMEM((NBITS,), jnp.int32),
        pltpu.VMEM((NBITS,), jnp.int32),
        pltpu.VMEM((NBITS, HIDDEN), jnp.float32),
        pltpu.VMEM((HIDDEN,), jnp.float32),
        pltpu.SemaphoreType.DMA,
    ],
)(_sc_bias_body)


def _add_body(bias_ref, hid_ref, out_ref):
    out_ref[...] = hid_ref[...] + bias_ref[...][:, :, None]


@functools.partial(jax.jit, donate_argnums=())
def kernel(hidden, msg, emb_table):
    bias = _sc_bias(msg.astype(jnp.int32), emb_table)

    grid = (BATCH // B_BLK, T // T_BLK)
    out = pl.pallas_call(
        _add_body,
        grid=grid,
        in_specs=[
            pl.BlockSpec((B_BLK, HIDDEN), lambda b, t: (b, 0)),
            pl.BlockSpec((B_BLK, HIDDEN, T_BLK), lambda b, t: (b, 0, t)),
        ],
        out_specs=pl.BlockSpec((B_BLK, HIDDEN, T_BLK), lambda b, t: (b, 0, t)),
        out_shape=jax.ShapeDtypeStruct((BATCH, HIDDEN, T), jnp.float32),
        compiler_params=pltpu.CompilerParams(
            dimension_semantics=("parallel", "parallel"),
        ),
    )(bias, hidden)
    return out


# speedup vs baseline: 1.1996x; 1.1996x over previous
import functools

import jax
import jax.numpy as jnp
from jax.experimental import pallas as pl
from jax.experimental.pallas import tpu as pltpu

NBITS = 16
HIDDEN = 128
BATCH = 32
T = 8192

B_BLK = 1
T_BLK = 8192


def _body(msg_ref, emb_ref, hid_ref, out_ref, bias_ref):
    b = pl.program_id(0)
    t = pl.program_id(1)

    @pl.when(jnp.logical_and(b == 0, t == 0))
    def _():
        even = emb_ref[:, 0, :]
        odd = emb_ref[:, 1, :]
        diff = odd - even
        base = jnp.sum(even, axis=0)
        bias_ref[...] = (
            jax.lax.dot(msg_ref[...], diff,
                        preferred_element_type=jnp.float32)
            + base[None, :]
        )

    blk_bias = bias_ref[pl.ds(b * B_BLK, B_BLK), :]
    out_ref[...] = hid_ref[...] + blk_bias[:, :, None]


@functools.partial(jax.jit, donate_argnums=())
def kernel(hidden, msg, emb_table):
    msg_f = msg.astype(jnp.float32)
    emb3 = emb_table.reshape(NBITS, 2, HIDDEN)

    grid = (BATCH // B_BLK, T // T_BLK)
    out = pl.pallas_call(
        _body,
        grid=grid,
        in_specs=[
            pl.BlockSpec((BATCH, NBITS), lambda b, t: (0, 0)),
            pl.BlockSpec((NBITS, 2, HIDDEN), lambda b, t: (0, 0, 0)),
            pl.BlockSpec((B_BLK, HIDDEN, T_BLK), lambda b, t: (b, 0, t)),
        ],
        out_specs=pl.BlockSpec((B_BLK, HIDDEN, T_BLK), lambda b, t: (b, 0, t)),
        out_shape=jax.ShapeDtypeStruct((BATCH, HIDDEN, T), jnp.float32),
        scratch_shapes=[pltpu.VMEM((BATCH, HIDDEN), jnp.float32)],
        compiler_params=pltpu.CompilerParams(
            dimension_semantics=("arbitrary", "arbitrary"),
        ),
    )(msg_f, emb3, hidden)
    return out


# speedup vs baseline: 1.2228x; 1.0193x over previous
import functools

import jax
import jax.numpy as jnp
from jax.experimental import pallas as pl
from jax.experimental.pallas import tpu as pltpu

NBITS = 16
HIDDEN = 128
BATCH = 32
T = 8192

B_BLK = 4
T_BLK = 4096


def _body(msg_ref, emb_ref, hid_ref, out_ref, bias_ref):
    b = pl.program_id(0)
    t = pl.program_id(1)

    @pl.when(jnp.logical_and(b == 0, t == 0))
    def _():
        even = emb_ref[:, 0, :]
        odd = emb_ref[:, 1, :]
        diff = odd - even
        base = jnp.sum(even, axis=0)
        bias_ref[...] = (
            jax.lax.dot(msg_ref[...], diff,
                        preferred_element_type=jnp.float32)
            + base[None, :]
        )

    blk_bias = bias_ref[pl.ds(b * B_BLK, B_BLK), :]
    out_ref[...] = hid_ref[...] + blk_bias[:, :, None]


@functools.partial(jax.jit, donate_argnums=())
def kernel(hidden, msg, emb_table):
    msg_f = msg.astype(jnp.float32)
    emb3 = emb_table.reshape(NBITS, 2, HIDDEN)

    grid = (BATCH // B_BLK, T // T_BLK)
    out = pl.pallas_call(
        _body,
        grid=grid,
        in_specs=[
            pl.BlockSpec((BATCH, NBITS), lambda b, t: (0, 0)),
            pl.BlockSpec((NBITS, 2, HIDDEN), lambda b, t: (0, 0, 0)),
            pl.BlockSpec((B_BLK, HIDDEN, T_BLK), lambda b, t: (b, 0, t)),
        ],
        out_specs=pl.BlockSpec((B_BLK, HIDDEN, T_BLK), lambda b, t: (b, 0, t)),
        out_shape=jax.ShapeDtypeStruct((BATCH, HIDDEN, T), jnp.float32),
        scratch_shapes=[pltpu.VMEM((BATCH, HIDDEN), jnp.float32)],
        compiler_params=pltpu.CompilerParams(
            dimension_semantics=("arbitrary", "arbitrary"),
        ),
    )(msg_f, emb3, hidden)
    return out


# speedup vs baseline: 1.2229x; 1.0001x over previous
import functools

import jax
import jax.numpy as jnp
from jax.experimental import pallas as pl
from jax.experimental.pallas import tpu as pltpu

NBITS = 16
HIDDEN = 128
BATCH = 32
T = 8192

B_BLK = 16
T_BLK = 1024


def _body(msg_ref, emb_ref, hid_ref, out_ref, bias_ref):
    b = pl.program_id(0)
    t = pl.program_id(1)

    @pl.when(jnp.logical_and(b == 0, t == 0))
    def _():
        even = emb_ref[:, 0, :]
        odd = emb_ref[:, 1, :]
        diff = odd - even
        base = jnp.sum(even, axis=0)
        bias_ref[...] = (
            jax.lax.dot(msg_ref[...], diff,
                        preferred_element_type=jnp.float32)
            + base[None, :]
        )

    blk_bias = bias_ref[pl.ds(b * B_BLK, B_BLK), :]
    out_ref[...] = hid_ref[...] + blk_bias[:, :, None]


@functools.partial(jax.jit, donate_argnums=())
def kernel(hidden, msg, emb_table):
    msg_f = msg.astype(jnp.float32)
    emb3 = emb_table.reshape(NBITS, 2, HIDDEN)

    grid = (BATCH // B_BLK, T // T_BLK)
    out = pl.pallas_call(
        _body,
        grid=grid,
        in_specs=[
            pl.BlockSpec((BATCH, NBITS), lambda b, t: (0, 0)),
            pl.BlockSpec((NBITS, 2, HIDDEN), lambda b, t: (0, 0, 0)),
            pl.BlockSpec((B_BLK, HIDDEN, T_BLK), lambda b, t: (b, 0, t)),
        ],
        out_specs=pl.BlockSpec((B_BLK, HIDDEN, T_BLK), lambda b, t: (b, 0, t)),
        out_shape=jax.ShapeDtypeStruct((BATCH, HIDDEN, T), jnp.float32),
        scratch_shapes=[pltpu.VMEM((BATCH, HIDDEN), jnp.float32)],
        compiler_params=pltpu.CompilerParams(
            dimension_semantics=("arbitrary", "arbitrary"),
        ),
    )(msg_f, emb3, hidden)
    return out
